# baseline (speedup 1.0000x reference)
import jax
import jax.numpy as jnp
from jax.experimental import pallas as pl

BATCH = 16384
DIM = 32
BLK = 2048


def _body(u_ref, i_ref, o_ref):
    o_ref[...] = jax.nn.sigmoid(jnp.sum(u_ref[...] * i_ref[...], axis=1))


def kernel(user, item, user_table, item_table):
    ue = jnp.take(user_table, user, axis=0)
    ie = jnp.take(item_table, item, axis=0)
    return pl.pallas_call(
        _body,
        out_shape=jax.ShapeDtypeStruct((BATCH,), jnp.float32),
        grid=(BATCH // BLK,),
        in_specs=[pl.BlockSpec((BLK, DIM), lambda g: (g, 0)),
                  pl.BlockSpec((BLK, DIM), lambda g: (g, 0))],
        out_specs=pl.BlockSpec((BLK,), lambda g: (g,)),
    )(ue, ie)
